# baseline (speedup 1.0000x reference)
import jax
import jax.numpy as jnp
from jax import lax
from jax.experimental import pallas as pl
from jax.experimental.pallas import tpu as pltpu

B = 256
C = 256
H = 25
W = 25
CROP = 7
ADD_U = 12
O = CROP * CROP


def _issue(s_ref, in_ref, buf_ref, sem, bb, slot):
    for i in range(CROP):
        y = s_ref[i, bb]
        pltpu.make_async_copy(
            in_ref.at[bb, :, y, :], buf_ref.at[slot, i], sem.at[slot]).start()


def _wait(s_ref, in_ref, buf_ref, sem, bb, slot):
    for i in range(CROP):
        y = s_ref[i, bb]
        pltpu.make_async_copy(
            in_ref.at[bb, :, y, :], buf_ref.at[slot, i], sem.at[slot]).wait()


def _tc_body(s_ref, in_ref, mf_ref, o_ref, buf_ref, sem):
    b = pl.program_id(0)
    slot = lax.rem(b, 2)

    @pl.when(b == 0)
    def _():
        _issue(s_ref, in_ref, buf_ref, sem, 0, 0)

    @pl.when(b + 1 < B)
    def _():
        _issue(s_ref, in_ref, buf_ref, sem, b + 1, lax.rem(b + 1, 2))

    _wait(s_ref, in_ref, buf_ref, sem, b, slot)

    vm3 = s_ref[CROP, b]
    u3 = s_ref[CROP + 1, b]
    po = lax.broadcasted_iota(jnp.int32, (W, O), 0)
    oo = lax.broadcasted_iota(jnp.int32, (W, O), 1)
    col = oo % CROP + vm3
    acc = jnp.zeros((C, O), jnp.float32)
    for i in range(CROP):
        xi = buf_ref[slot, i]
        ok = (oo // CROP == i) & (col >= 0) & (u3 + i <= H - 1)
        sel = jnp.where(ok & (po == col), 1.0, 0.0)
        acc = acc + lax.dot_general(xi, sel, (((1,), (0,)), ((), ())),
                                    preferred_element_type=jnp.float32)
    o_ref[0] = acc * mf_ref[0][None, :]


def kernel(input_tensor, center_positions, mask, crop_mask):
    r = center_positions[:, 0].astype(jnp.int32)
    q = center_positions[:, 1].astype(jnp.int32)
    u3 = r - q // 2 + ADD_U - (CROP - 1) // 2
    vm3 = q - (CROP - 1) // 2
    rows = [jnp.clip(u3 + i, 0, H - 1) for i in range(CROP)]
    scals = jnp.stack(rows + [vm3, u3]).astype(jnp.int32)

    mask_factor = jnp.where(
        mask != 0, crop_mask, jnp.ones_like(crop_mask)).astype(jnp.float32)
    mf = mask_factor.reshape(1, O)

    grid_spec = pltpu.PrefetchScalarGridSpec(
        num_scalar_prefetch=1,
        grid=(B,),
        in_specs=[
            pl.BlockSpec(memory_space=pl.ANY),
            pl.BlockSpec((1, O), lambda b, s: (0, 0)),
        ],
        out_specs=pl.BlockSpec((1, C, O), lambda b, s: (b, 0, 0)),
        scratch_shapes=[
            pltpu.VMEM((2, CROP, C, W), jnp.float32),
            pltpu.SemaphoreType.DMA((2,)),
        ],
    )
    out = pl.pallas_call(
        _tc_body,
        grid_spec=grid_spec,
        out_shape=jax.ShapeDtypeStruct((B, C, O), jnp.float32),
    )(scals, input_tensor, mf)
    return (out.reshape(B, C, CROP, CROP), crop_mask)


# speedup vs baseline: 2.6975x; 2.6975x over previous
import jax
import jax.numpy as jnp
from jax import lax
from jax.experimental import pallas as pl
from jax.experimental.pallas import tpu as pltpu

B = 256
C = 256
H = 25
W = 25
CROP = 7
ADD_U = 12
P = H * W
O = CROP * CROP
G = 4


def _tc_body(s_ref, x_ref, mf_ref, o_ref):
    g0 = pl.program_id(0) * G
    p = lax.broadcasted_iota(jnp.int32, (P, O), 0)
    o = lax.broadcasted_iota(jnp.int32, (P, O), 1)
    base_t = (o // CROP) * W + (o % CROP)
    for k in range(G):
        b = g0 + k
        u3 = s_ref[0, b]
        vm3 = s_ref[1, b]
        t = base_t + (u3 * W + vm3)
        ok = (o % CROP) + vm3 >= 0
        sel = jnp.where((p == t) & ok, 1.0, 0.0)
        res = lax.dot_general(x_ref[k], sel, (((1,), (0,)), ((), ())),
                              preferred_element_type=jnp.float32)
        o_ref[k] = res * mf_ref[0][None, :]


def kernel(input_tensor, center_positions, mask, crop_mask):
    r = center_positions[:, 0].astype(jnp.int32)
    q = center_positions[:, 1].astype(jnp.int32)
    u3 = r - q // 2 + ADD_U - (CROP - 1) // 2
    vm3 = q - (CROP - 1) // 2
    scals = jnp.stack([u3, vm3]).astype(jnp.int32)

    mask_factor = jnp.where(
        mask != 0, crop_mask, jnp.ones_like(crop_mask)).astype(jnp.float32)
    mf = mask_factor.reshape(1, O)

    grid_spec = pltpu.PrefetchScalarGridSpec(
        num_scalar_prefetch=1,
        grid=(B // G,),
        in_specs=[
            pl.BlockSpec((G, C, P), lambda b, s: (b, 0, 0)),
            pl.BlockSpec((1, O), lambda b, s: (0, 0)),
        ],
        out_specs=pl.BlockSpec((G, C, O), lambda b, s: (b, 0, 0)),
    )
    out = pl.pallas_call(
        _tc_body,
        grid_spec=grid_spec,
        out_shape=jax.ShapeDtypeStruct((B, C, O), jnp.float32),
    )(scals, input_tensor.reshape(B, C, P), mf)
    return (out.reshape(B, C, CROP, CROP), crop_mask)


# speedup vs baseline: 2.9060x; 1.0773x over previous
import jax
import jax.numpy as jnp
from jax import lax
from jax.experimental import pallas as pl
from jax.experimental.pallas import tpu as pltpu

B = 256
C = 256
H = 25
W = 25
CROP = 7
ADD_U = 12
P = H * W
O = CROP * CROP
G = 8


def _tc_body(s_ref, x_ref, mf_ref, o_ref):
    g0 = pl.program_id(0) * G
    p = lax.broadcasted_iota(jnp.int32, (P, O), 0)
    o = lax.broadcasted_iota(jnp.int32, (P, O), 1)
    base_t = (o // CROP) * W + (o % CROP)
    for k in range(G):
        b = g0 + k
        u3 = s_ref[0, b]
        vm3 = s_ref[1, b]
        t = base_t + (u3 * W + vm3)
        ok = (o % CROP) + vm3 >= 0
        sel = jnp.where((p == t) & ok, 1.0, 0.0)
        res = lax.dot_general(x_ref[k], sel, (((1,), (0,)), ((), ())),
                              preferred_element_type=jnp.float32)
        o_ref[k] = res * mf_ref[0][None, :]


def kernel(input_tensor, center_positions, mask, crop_mask):
    r = center_positions[:, 0].astype(jnp.int32)
    q = center_positions[:, 1].astype(jnp.int32)
    u3 = r - q // 2 + ADD_U - (CROP - 1) // 2
    vm3 = q - (CROP - 1) // 2
    scals = jnp.stack([u3, vm3]).astype(jnp.int32)

    mask_factor = jnp.where(
        mask != 0, crop_mask, jnp.ones_like(crop_mask)).astype(jnp.float32)
    mf = mask_factor.reshape(1, O)

    grid_spec = pltpu.PrefetchScalarGridSpec(
        num_scalar_prefetch=1,
        grid=(B // G,),
        in_specs=[
            pl.BlockSpec((G, C, P), lambda b, s: (b, 0, 0)),
            pl.BlockSpec((1, O), lambda b, s: (0, 0)),
        ],
        out_specs=pl.BlockSpec((G, C, O), lambda b, s: (b, 0, 0)),
    )
    out = pl.pallas_call(
        _tc_body,
        grid_spec=grid_spec,
        out_shape=jax.ShapeDtypeStruct((B, C, O), jnp.float32),
    )(scals, input_tensor.reshape(B, C, P), mf)
    return (out.reshape(B, C, CROP, CROP), crop_mask)


# speedup vs baseline: 2.9631x; 1.0196x over previous
import jax
import jax.numpy as jnp
from jax import lax
from jax.experimental import pallas as pl
from jax.experimental.pallas import tpu as pltpu

B = 256
C = 256
H = 25
W = 25
CROP = 7
ADD_U = 12
P = H * W
O = CROP * CROP
G = 16


def _tc_body(s_ref, x_ref, mf_ref, o_ref):
    g0 = pl.program_id(0) * G
    p = lax.broadcasted_iota(jnp.int32, (P, O), 0)
    o = lax.broadcasted_iota(jnp.int32, (P, O), 1)
    base_t = (o // CROP) * W + (o % CROP)
    for k in range(G):
        b = g0 + k
        u3 = s_ref[0, b]
        vm3 = s_ref[1, b]
        t = base_t + (u3 * W + vm3)
        ok = (o % CROP) + vm3 >= 0
        sel = jnp.where((p == t) & ok, 1.0, 0.0)
        res = lax.dot_general(x_ref[k], sel, (((1,), (0,)), ((), ())),
                              preferred_element_type=jnp.float32)
        o_ref[k] = res * mf_ref[0][None, :]


def kernel(input_tensor, center_positions, mask, crop_mask):
    r = center_positions[:, 0].astype(jnp.int32)
    q = center_positions[:, 1].astype(jnp.int32)
    u3 = r - q // 2 + ADD_U - (CROP - 1) // 2
    vm3 = q - (CROP - 1) // 2
    scals = jnp.stack([u3, vm3]).astype(jnp.int32)

    mask_factor = jnp.where(
        mask != 0, crop_mask, jnp.ones_like(crop_mask)).astype(jnp.float32)
    mf = mask_factor.reshape(1, O)

    grid_spec = pltpu.PrefetchScalarGridSpec(
        num_scalar_prefetch=1,
        grid=(B // G,),
        in_specs=[
            pl.BlockSpec((G, C, P), lambda b, s: (b, 0, 0)),
            pl.BlockSpec((1, O), lambda b, s: (0, 0)),
        ],
        out_specs=pl.BlockSpec((G, C, O), lambda b, s: (b, 0, 0)),
    )
    out = pl.pallas_call(
        _tc_body,
        grid_spec=grid_spec,
        out_shape=jax.ShapeDtypeStruct((B, C, O), jnp.float32),
    )(scals, input_tensor.reshape(B, C, P), mf)
    return (out.reshape(B, C, CROP, CROP), crop_mask)


# speedup vs baseline: 11.0395x; 3.7256x over previous
import jax
import jax.numpy as jnp
from jax import lax
from jax.experimental import pallas as pl
from jax.experimental.pallas import tpu as pltpu

B = 256
C = 256
H = 25
W = 25
CROP = 7
ADD_U = 12
GB = 8
NS = B // GB


def _tc_body(s_ref, x_ref, mf_ref, o_ref, sc_ref):
    s = pl.program_id(0)

    @pl.when(s == 0)
    def _():
        sc_ref[...] = jnp.zeros((10, 10, GB, C), jnp.float32)

    bidx = lax.broadcasted_iota(jnp.int32, (CROP, CROP, GB, C), 2)
    zeros7 = jnp.zeros((CROP, CROP, GB, C), jnp.float32)
    acc = zeros7
    for k in range(GB):
        b = s * GB + k
        yc0 = s_ref[0, b]
        xc0 = s_ref[1, b]
        ro = s_ref[2, b]
        co = s_ref[3, b]
        xw = x_ref[pl.ds(yc0, CROP), pl.ds(xc0, CROP), :, :]
        sc_ref[pl.ds(ro, CROP), pl.ds(co, CROP), :, :] = xw
        win = sc_ref[3:3 + CROP, 0:CROP, :, :]
        acc = jnp.where(bidx == k, win, acc)
        sc_ref[pl.ds(ro, CROP), pl.ds(co, CROP), :, :] = zeros7
    o_ref[...] = acc * mf_ref[...]


def kernel(input_tensor, center_positions, mask, crop_mask):
    r = center_positions[:, 0].astype(jnp.int32)
    q = center_positions[:, 1].astype(jnp.int32)
    u3 = r - q // 2 + ADD_U - (CROP - 1) // 2
    vm3 = q - (CROP - 1) // 2
    yc0 = jnp.clip(u3, 0, H - CROP)
    xc0 = jnp.clip(vm3, 0, W - CROP)
    rowoff = 3 - (u3 - yc0)
    coloff = xc0 - vm3
    scals = jnp.stack([yc0, xc0, rowoff, coloff]).astype(jnp.int32)

    mask_factor = jnp.where(
        mask != 0, crop_mask, jnp.ones_like(crop_mask)).astype(jnp.float32)
    mf4 = jnp.broadcast_to(mask_factor[:, :, None, None], (CROP, CROP, GB, C))

    grid_spec = pltpu.PrefetchScalarGridSpec(
        num_scalar_prefetch=1,
        grid=(NS,),
        in_specs=[
            pl.BlockSpec((H, W, GB, C), lambda s, sc: (0, 0, s, 0)),
            pl.BlockSpec((CROP, CROP, GB, C), lambda s, sc: (0, 0, 0, 0)),
        ],
        out_specs=pl.BlockSpec((CROP, CROP, GB, C), lambda s, sc: (0, 0, s, 0)),
        scratch_shapes=[
            pltpu.VMEM((10, 10, GB, C), jnp.float32),
        ],
    )
    out_t = pl.pallas_call(
        _tc_body,
        grid_spec=grid_spec,
        out_shape=jax.ShapeDtypeStruct((CROP, CROP, B, C), jnp.float32),
    )(scals, input_tensor.transpose(2, 3, 0, 1), mf4)
    return (out_t.transpose(2, 3, 0, 1), crop_mask)
